# Initial kernel scaffold; baseline (speedup 1.0000x reference)
import jax
import jax.numpy as jnp
from jax.experimental import pallas as pl


def kernel(x, adj, W, b, gamma, beta):
    raise NotImplementedError("write your pallas kernel here")


# speedup vs baseline: 1.0522x; 1.0522x over previous
import jax
import jax.numpy as jnp
from jax.experimental import pallas as pl
from jax.experimental.pallas import tpu as pltpu

_N = 10000
_F = 128
_TM = 400
_G = _N // _TM
_EPS = 1e-5


def _gcn_kernel(x_ref, adj_ref, w_ref, b_ref, gamma_ref, beta_ref,
                out_ref, support_ref):
    i = pl.program_id(0)

    @pl.when(i == 0)
    def _():
        support_ref[...] = jnp.dot(x_ref[...], w_ref[...],
                                   preferred_element_type=jnp.float32)

    @pl.when(i < _G)
    def _():
        h = jnp.dot(adj_ref[...], support_ref[...],
                    preferred_element_type=jnp.float32)
        out_ref[pl.ds(i * _TM, _TM), :] = h + b_ref[...]

    @pl.when(i == _G)
    def _():
        h = out_ref[...]
        mean = jnp.mean(h, axis=0, keepdims=True)
        var = jnp.mean((h - mean) ** 2, axis=0, keepdims=True)
        inv = jax.lax.rsqrt(var + _EPS)
        out_ref[...] = jnp.maximum(
            (h - mean) * inv * gamma_ref[...] + beta_ref[...], 0.0)


def kernel(x, adj, W, b, gamma, beta):
    b2 = b.reshape(1, _F)
    gamma2 = gamma.reshape(1, _F)
    beta2 = beta.reshape(1, _F)
    return pl.pallas_call(
        _gcn_kernel,
        grid=(_G + 1,),
        in_specs=[
            pl.BlockSpec((_N, _F), lambda i: (0, 0)),
            pl.BlockSpec((_TM, _N), lambda i: (jnp.minimum(i, _G - 1), 0)),
            pl.BlockSpec((_F, _F), lambda i: (0, 0)),
            pl.BlockSpec((1, _F), lambda i: (0, 0)),
            pl.BlockSpec((1, _F), lambda i: (0, 0)),
            pl.BlockSpec((1, _F), lambda i: (0, 0)),
        ],
        out_specs=pl.BlockSpec((_N, _F), lambda i: (0, 0)),
        out_shape=jax.ShapeDtypeStruct((_N, _F), jnp.float32),
        scratch_shapes=[pltpu.VMEM((_N, _F), jnp.float32)],
    )(x, adj, W, b2, gamma2, beta2)


# speedup vs baseline: 1.0563x; 1.0038x over previous
import jax
import jax.numpy as jnp
from jax.experimental import pallas as pl
from jax.experimental.pallas import tpu as pltpu

_N = 10000
_F = 128
_TM = 200
_G = _N // _TM
_EPS = 1e-5


def _gcn_kernel(x_ref, adj_ref, w_ref, b_ref, gamma_ref, beta_ref,
                out_ref, support_ref):
    i = pl.program_id(0)

    @pl.when(i == 0)
    def _():
        support_ref[...] = jnp.dot(x_ref[...], w_ref[...],
                                   preferred_element_type=jnp.float32)

    @pl.when(i < _G)
    def _():
        h = jnp.dot(adj_ref[...], support_ref[...],
                    preferred_element_type=jnp.float32)
        out_ref[pl.ds(i * _TM, _TM), :] = h + b_ref[...]

    @pl.when(i == _G)
    def _():
        h = out_ref[...]
        mean = jnp.mean(h, axis=0, keepdims=True)
        var = jnp.mean((h - mean) ** 2, axis=0, keepdims=True)
        inv = jax.lax.rsqrt(var + _EPS)
        out_ref[...] = jnp.maximum(
            (h - mean) * inv * gamma_ref[...] + beta_ref[...], 0.0)


def kernel(x, adj, W, b, gamma, beta):
    b2 = b.reshape(1, _F)
    gamma2 = gamma.reshape(1, _F)
    beta2 = beta.reshape(1, _F)
    return pl.pallas_call(
        _gcn_kernel,
        grid=(_G + 1,),
        in_specs=[
            pl.BlockSpec((_N, _F), lambda i: (0, 0)),
            pl.BlockSpec((_TM, _N), lambda i: (jnp.minimum(i, _G - 1), 0)),
            pl.BlockSpec((_F, _F), lambda i: (0, 0)),
            pl.BlockSpec((1, _F), lambda i: (0, 0)),
            pl.BlockSpec((1, _F), lambda i: (0, 0)),
            pl.BlockSpec((1, _F), lambda i: (0, 0)),
        ],
        out_specs=pl.BlockSpec((_N, _F), lambda i: (0, 0)),
        out_shape=jax.ShapeDtypeStruct((_N, _F), jnp.float32),
        scratch_shapes=[pltpu.VMEM((_N, _F), jnp.float32)],
    )(x, adj, W, b2, gamma2, beta2)
